# Initial kernel scaffold; baseline (speedup 1.0000x reference)
import jax
import jax.numpy as jnp
from jax.experimental import pallas as pl


def kernel(states, action, We1, be1, We2, be2, ge, bte, We3, be3, Wn1, bn1, Wn2, bn2, gn, btn, Wn3, bn3):
    raise NotImplementedError("write your pallas kernel here")


# speedup vs baseline: 3.6924x; 3.6924x over previous
import functools

import jax
import jax.numpy as jnp
from jax.experimental import pallas as pl


def _gnn_block_kernel(
    xs_ref, a_ref, w1uv_ref, be1_ref, we2t_ref, be2_ref, ge_ref, bte_ref,
    we3t_ref, be3_ref, wn1at_ref, wn1bt_ref, wn1ct_ref, bn1_ref, wn2t_ref,
    bn2_ref, gn_ref, btn_ref, wn3t_ref, bn3_ref, out_ref, *, G, O, D, H, A,
):
    f32 = jnp.float32
    x = xs_ref[:]

    uv = jnp.dot(x, w1uv_ref[:], preferred_element_type=f32)
    u = uv[:, :H]
    v = uv[:, H:]
    u4 = u.reshape(G, O, 1, H)
    v4 = v.reshape(G, 1, O, H)
    p = jnp.maximum(u4 + v4 + be1_ref[:].reshape(1, 1, 1, H), 0.0)
    p = p.reshape(G * O * O, H)

    h = jnp.dot(p, we2t_ref[:], preferred_element_type=f32) + be2_ref[:]
    m = jnp.mean(h, axis=-1, keepdims=True)
    hc = h - m
    var = jnp.mean(hc * hc, axis=-1, keepdims=True)
    h = hc * jax.lax.rsqrt(var + 1e-5) * ge_ref[:] + bte_ref[:]
    h = jnp.maximum(h, 0.0)
    e3 = jnp.dot(h, we3t_ref[:], preferred_element_type=f32) + be3_ref[:]

    e4 = e3.reshape(G, O, O, H)
    ii = jax.lax.broadcasted_iota(jnp.int32, (1, O, O, 1), 1)
    jj = jax.lax.broadcasted_iota(jnp.int32, (1, O, O, 1), 2)
    mask = (ii != jj).astype(f32)
    agg = jnp.sum(e4 * mask, axis=2).reshape(G * O, H)

    a = a_ref[:]
    onehot = (a == jax.lax.broadcasted_iota(jnp.int32, (1, A), 1)).astype(f32)
    acth = jnp.dot(onehot, wn1bt_ref[:], preferred_element_type=f32)
    acth = jnp.broadcast_to(acth.reshape(G, 1, H), (G, O, H)).reshape(G * O, H)
    t = (jnp.dot(x, wn1at_ref[:], preferred_element_type=f32)
         + acth
         + jnp.dot(agg, wn1ct_ref[:], preferred_element_type=f32)
         + bn1_ref[:])
    t = jnp.maximum(t, 0.0)
    h = jnp.dot(t, wn2t_ref[:], preferred_element_type=f32) + bn2_ref[:]
    m = jnp.mean(h, axis=-1, keepdims=True)
    hc = h - m
    var = jnp.mean(hc * hc, axis=-1, keepdims=True)
    h = hc * jax.lax.rsqrt(var + 1e-5) * gn_ref[:] + btn_ref[:]
    h = jnp.maximum(h, 0.0)
    out_ref[:] = jnp.dot(h, wn3t_ref[:], preferred_element_type=f32) + bn3_ref[:]


@functools.partial(jax.jit, static_argnames=("G", "interpret"))
def _run(states, action, We1, be1, We2, be2, ge, bte, We3, be3,
         Wn1, bn1, Wn2, bn2, gn, btn, Wn3, bn3, *, G=128, interpret=False):
    Bv, O, D = states.shape
    H = We1.shape[0]
    A = Wn1.shape[1] - H - D
    assert Bv % G == 0
    grid = Bv // G

    xs = states.reshape(Bv * O, D)
    a2 = action.astype(jnp.int32).reshape(Bv, 1)
    w1uv = jnp.concatenate([We1[:, :D].T, We1[:, D:].T], axis=1)
    we2t = We2.T
    we3t = We3.T
    wn1at = Wn1[:, :D].T
    wn1bt = Wn1[:, D:D + A].T
    wn1ct = Wn1[:, D + A:].T
    wn2t = Wn2.T
    wn3t = Wn3.T
    row = lambda z: z.reshape(1, -1)

    full = lambda arr: pl.BlockSpec(arr.shape, lambda i: (0,) * arr.ndim)
    kern = functools.partial(_gnn_block_kernel, G=G, O=O, D=D, H=H, A=A)
    out = pl.pallas_call(
        kern,
        grid=(grid,),
        in_specs=[
            pl.BlockSpec((G * O, D), lambda i: (i, 0)),
            pl.BlockSpec((G, 1), lambda i: (i, 0)),
            full(w1uv), full(row(be1)), full(we2t), full(row(be2)),
            full(row(ge)), full(row(bte)), full(we3t), full(row(be3)),
            full(wn1at), full(wn1bt), full(wn1ct), full(row(bn1)),
            full(wn2t), full(row(bn2)), full(row(gn)), full(row(btn)),
            full(wn3t), full(row(bn3)),
        ],
        out_specs=pl.BlockSpec((G * O, D), lambda i: (i, 0)),
        out_shape=jax.ShapeDtypeStruct((Bv * O, D), jnp.float32),
        interpret=interpret,
    )(
        xs, a2, w1uv, row(be1), we2t, row(be2), row(ge), row(bte),
        we3t, row(be3), wn1at, wn1bt, wn1ct, row(bn1), wn2t, row(bn2),
        row(gn), row(btn), wn3t, row(bn3),
    )
    return out.reshape(Bv, O, D)


def kernel(states, action, We1, be1, We2, be2, ge, bte, We3, be3,
           Wn1, bn1, Wn2, bn2, gn, btn, Wn3, bn3):
    return _run(states, action, We1, be1, We2, be2, ge, bte, We3, be3,
                Wn1, bn1, Wn2, bn2, gn, btn, Wn3, bn3)


# speedup vs baseline: 11.0365x; 2.9889x over previous
import functools

import jax
import jax.numpy as jnp
from jax.experimental import pallas as pl


def _gnn_block_kernel(
    xs_ref, a_ref, w1uv_ref, be1_ref, we2t_ref, be2_ref, ge_ref, bte_ref,
    we3t_ref, be3_ref, wn1at_ref, wn1bt_ref, wn1ct_ref, bn1_ref, wn2t_ref,
    bn2_ref, gn_ref, btn_ref, wn3t_ref, bn3_ref, out_ref, *, G, O, D, H, A,
):
    f32 = jnp.float32
    x = xs_ref[:]
    x2 = x.reshape(O * G, D)

    uv = jnp.dot(x2, w1uv_ref[:], preferred_element_type=f32)
    u = uv[:, :H].reshape(O, 1, G, H)
    v = uv[:, H:].reshape(1, O, G, H)
    p = jnp.maximum(u + v + be1_ref[:].reshape(1, 1, 1, H), 0.0)
    p = p.reshape(O * O * G, H)

    h = jnp.dot(p, we2t_ref[:], preferred_element_type=f32) + be2_ref[:]
    m = jnp.mean(h, axis=-1, keepdims=True)
    hc = h - m
    var = jnp.mean(hc * hc, axis=-1, keepdims=True)
    h = hc * jax.lax.rsqrt(var + 1e-5) * ge_ref[:] + bte_ref[:]
    h = jnp.maximum(h, 0.0)
    e3 = jnp.dot(h, we3t_ref[:], preferred_element_type=f32) + be3_ref[:]

    e4 = e3.reshape(O, O, G, H)
    diag = jnp.stack([e4[i, i] for i in range(O)], axis=0)
    agg = (jnp.sum(e4, axis=1) - diag).reshape(O * G, H)

    a = a_ref[:]
    onehot = (a == jax.lax.broadcasted_iota(jnp.int32, (1, A), 1)).astype(f32)
    acth = jnp.dot(onehot, wn1bt_ref[:], preferred_element_type=f32)
    acth = jnp.broadcast_to(acth.reshape(1, G, H), (O, G, H)).reshape(O * G, H)
    t = (jnp.dot(x2, wn1at_ref[:], preferred_element_type=f32)
         + acth
         + jnp.dot(agg, wn1ct_ref[:], preferred_element_type=f32)
         + bn1_ref[:])
    t = jnp.maximum(t, 0.0)
    h = jnp.dot(t, wn2t_ref[:], preferred_element_type=f32) + bn2_ref[:]
    m = jnp.mean(h, axis=-1, keepdims=True)
    hc = h - m
    var = jnp.mean(hc * hc, axis=-1, keepdims=True)
    h = hc * jax.lax.rsqrt(var + 1e-5) * gn_ref[:] + btn_ref[:]
    h = jnp.maximum(h, 0.0)
    out = jnp.dot(h, wn3t_ref[:], preferred_element_type=f32) + bn3_ref[:]
    out_ref[:] = out.reshape(O, G, D)


@functools.partial(jax.jit, static_argnames=("G", "interpret"))
def _run(states, action, We1, be1, We2, be2, ge, bte, We3, be3,
         Wn1, bn1, Wn2, bn2, gn, btn, Wn3, bn3, *, G=128, interpret=False):
    Bv, O, D = states.shape
    H = We1.shape[0]
    A = Wn1.shape[1] - H - D
    assert Bv % G == 0
    grid = Bv // G

    xs = states.transpose(1, 0, 2)
    a2 = action.astype(jnp.int32).reshape(Bv, 1)
    w1uv = jnp.concatenate([We1[:, :D].T, We1[:, D:].T], axis=1)
    we2t = We2.T
    we3t = We3.T
    wn1at = Wn1[:, :D].T
    wn1bt = Wn1[:, D:D + A].T
    wn1ct = Wn1[:, D + A:].T
    wn2t = Wn2.T
    wn3t = Wn3.T
    row = lambda z: z.reshape(1, -1)

    full = lambda arr: pl.BlockSpec(arr.shape, lambda i: (0,) * arr.ndim)
    kern = functools.partial(_gnn_block_kernel, G=G, O=O, D=D, H=H, A=A)
    out = pl.pallas_call(
        kern,
        grid=(grid,),
        in_specs=[
            pl.BlockSpec((O, G, D), lambda i: (0, i, 0)),
            pl.BlockSpec((G, 1), lambda i: (i, 0)),
            full(w1uv), full(row(be1)), full(we2t), full(row(be2)),
            full(row(ge)), full(row(bte)), full(we3t), full(row(be3)),
            full(wn1at), full(wn1bt), full(wn1ct), full(row(bn1)),
            full(wn2t), full(row(bn2)), full(row(gn)), full(row(btn)),
            full(wn3t), full(row(bn3)),
        ],
        out_specs=pl.BlockSpec((O, G, D), lambda i: (0, i, 0)),
        out_shape=jax.ShapeDtypeStruct((O, Bv, D), jnp.float32),
        interpret=interpret,
    )(
        xs, a2, w1uv, row(be1), we2t, row(be2), row(ge), row(bte),
        we3t, row(be3), wn1at, wn1bt, wn1ct, row(bn1), wn2t, row(bn2),
        row(gn), row(btn), wn3t, row(bn3),
    )
    return out.transpose(1, 0, 2)


def kernel(states, action, We1, be1, We2, be2, ge, bte, We3, be3,
           Wn1, bn1, Wn2, bn2, gn, btn, Wn3, bn3):
    return _run(states, action, We1, be1, We2, be2, ge, bte, We3, be3,
                Wn1, bn1, Wn2, bn2, gn, btn, Wn3, bn3)


# speedup vs baseline: 13.4558x; 1.2192x over previous
import functools

import jax
import jax.numpy as jnp
from jax.experimental import pallas as pl


def _gnn_block_kernel(
    xs_ref, a_ref, w1uv_ref, be1_ref, we2t_ref, be2_ref, ge_ref, bte_ref,
    we3t_ref, be3_ref, wn1at_ref, wn1bt_ref, wn1ct_ref, bn1_ref, wn2t_ref,
    bn2_ref, gn_ref, btn_ref, wn3t_ref, bn3_ref, out_ref, *, G, O, D, H, A,
):
    f32 = jnp.float32
    x = xs_ref[:]
    x2 = x.reshape(O * G, D)

    uv = jnp.dot(x2, w1uv_ref[:], preferred_element_type=f32)
    u = uv[:, :H].reshape(O, 1, G, H)
    v = uv[:, H:].reshape(1, O, G, H)
    p = jnp.maximum(u + v + be1_ref[:].reshape(1, 1, 1, H), 0.0)
    p = p.reshape(O * O * G, H)

    h = jnp.dot(p, we2t_ref[:], preferred_element_type=f32) + be2_ref[:]
    m = jnp.mean(h, axis=-1, keepdims=True)
    hc = h - m
    var = jnp.mean(hc * hc, axis=-1, keepdims=True)
    h = hc * jax.lax.rsqrt(var + 1e-5) * ge_ref[:] + bte_ref[:]
    h = jnp.maximum(h, 0.0)
    e3 = jnp.dot(h, we3t_ref[:], preferred_element_type=f32) + be3_ref[:]

    e4 = e3.reshape(O, O, G, H)
    diag = jnp.stack([e4[i, i] for i in range(O)], axis=0)
    agg = (jnp.sum(e4, axis=1) - diag).reshape(O * G, H)

    a = a_ref[:]
    onehot = (a == jax.lax.broadcasted_iota(jnp.int32, (1, A), 1)).astype(f32)
    acth = jnp.dot(onehot, wn1bt_ref[:], preferred_element_type=f32)
    acth = jnp.broadcast_to(acth.reshape(1, G, H), (O, G, H)).reshape(O * G, H)
    t = (jnp.dot(x2, wn1at_ref[:], preferred_element_type=f32)
         + acth
         + jnp.dot(agg, wn1ct_ref[:], preferred_element_type=f32)
         + bn1_ref[:])
    t = jnp.maximum(t, 0.0)
    h = jnp.dot(t, wn2t_ref[:], preferred_element_type=f32) + bn2_ref[:]
    m = jnp.mean(h, axis=-1, keepdims=True)
    hc = h - m
    var = jnp.mean(hc * hc, axis=-1, keepdims=True)
    h = hc * jax.lax.rsqrt(var + 1e-5) * gn_ref[:] + btn_ref[:]
    h = jnp.maximum(h, 0.0)
    out = jnp.dot(h, wn3t_ref[:], preferred_element_type=f32) + bn3_ref[:]
    out_ref[:] = out.reshape(O, G, D)


@functools.partial(jax.jit, static_argnames=("G", "interpret"))
def _run(states, action, We1, be1, We2, be2, ge, bte, We3, be3,
         Wn1, bn1, Wn2, bn2, gn, btn, Wn3, bn3, *, G=256, interpret=False):
    Bv, O, D = states.shape
    H = We1.shape[0]
    A = Wn1.shape[1] - H - D
    assert Bv % G == 0
    grid = Bv // G

    xs = states.transpose(1, 0, 2)
    a2 = action.astype(jnp.int32).reshape(Bv, 1)
    w1uv = jnp.concatenate([We1[:, :D].T, We1[:, D:].T], axis=1)
    we2t = We2.T
    we3t = We3.T
    wn1at = Wn1[:, :D].T
    wn1bt = Wn1[:, D:D + A].T
    wn1ct = Wn1[:, D + A:].T
    wn2t = Wn2.T
    wn3t = Wn3.T
    row = lambda z: z.reshape(1, -1)

    full = lambda arr: pl.BlockSpec(arr.shape, lambda i: (0,) * arr.ndim)
    kern = functools.partial(_gnn_block_kernel, G=G, O=O, D=D, H=H, A=A)
    out = pl.pallas_call(
        kern,
        grid=(grid,),
        in_specs=[
            pl.BlockSpec((O, G, D), lambda i: (0, i, 0)),
            pl.BlockSpec((G, 1), lambda i: (i, 0)),
            full(w1uv), full(row(be1)), full(we2t), full(row(be2)),
            full(row(ge)), full(row(bte)), full(we3t), full(row(be3)),
            full(wn1at), full(wn1bt), full(wn1ct), full(row(bn1)),
            full(wn2t), full(row(bn2)), full(row(gn)), full(row(btn)),
            full(wn3t), full(row(bn3)),
        ],
        out_specs=pl.BlockSpec((O, G, D), lambda i: (0, i, 0)),
        out_shape=jax.ShapeDtypeStruct((O, Bv, D), jnp.float32),
        interpret=interpret,
    )(
        xs, a2, w1uv, row(be1), we2t, row(be2), row(ge), row(bte),
        we3t, row(be3), wn1at, wn1bt, wn1ct, row(bn1), wn2t, row(bn2),
        row(gn), row(btn), wn3t, row(bn3),
    )
    return out.transpose(1, 0, 2)


def kernel(states, action, We1, be1, We2, be2, ge, bte, We3, be3,
           Wn1, bn1, Wn2, bn2, gn, btn, Wn3, bn3):
    return _run(states, action, We1, be1, We2, be2, ge, bte, We3, be3,
                Wn1, bn1, Wn2, bn2, gn, btn, Wn3, bn3)


# speedup vs baseline: 14.3957x; 1.0699x over previous
import functools

import jax
import jax.numpy as jnp
from jax.experimental import pallas as pl


def _gnn_block_kernel(
    xs_ref, a_ref, w1uv_ref, be1_ref, we2t_ref, be2_ref, ge_ref, bte_ref,
    we3t_ref, be3_ref, wn1at_ref, wn1bt_ref, wn1ct_ref, bn1_ref, wn2t_ref,
    bn2_ref, gn_ref, btn_ref, wn3t_ref, bn3_ref, out_ref, *, G, O, D, H, A,
):
    f32 = jnp.float32
    x = xs_ref[:]
    x2 = x.reshape(O * G, D)

    uv = jnp.dot(x2, w1uv_ref[:], preferred_element_type=f32)
    u = uv[:, :H].reshape(O, 1, G, H)
    v = uv[:, H:].reshape(1, O, G, H)
    p = jnp.maximum(u + v + be1_ref[:].reshape(1, 1, 1, H), 0.0)
    p = p.reshape(O * O * G, H)

    h = jnp.dot(p, we2t_ref[:], preferred_element_type=f32) + be2_ref[:]
    m = jnp.mean(h, axis=-1, keepdims=True)
    hc = h - m
    var = jnp.mean(hc * hc, axis=-1, keepdims=True)
    h = hc * jax.lax.rsqrt(var + 1e-5) * ge_ref[:] + bte_ref[:]
    h = jnp.maximum(h, 0.0)
    e3 = jnp.dot(h, we3t_ref[:], preferred_element_type=f32) + be3_ref[:]

    e4 = e3.reshape(O, O, G, H)
    diag = jnp.stack([e4[i, i] for i in range(O)], axis=0)
    agg = (jnp.sum(e4, axis=1) - diag).reshape(O * G, H)

    a = a_ref[:]
    onehot = (a == jax.lax.broadcasted_iota(jnp.int32, (1, A), 1)).astype(f32)
    acth = jnp.dot(onehot, wn1bt_ref[:], preferred_element_type=f32)
    acth = jnp.broadcast_to(acth.reshape(1, G, H), (O, G, H)).reshape(O * G, H)
    t = (jnp.dot(x2, wn1at_ref[:], preferred_element_type=f32)
         + acth
         + jnp.dot(agg, wn1ct_ref[:], preferred_element_type=f32)
         + bn1_ref[:])
    t = jnp.maximum(t, 0.0)
    h = jnp.dot(t, wn2t_ref[:], preferred_element_type=f32) + bn2_ref[:]
    m = jnp.mean(h, axis=-1, keepdims=True)
    hc = h - m
    var = jnp.mean(hc * hc, axis=-1, keepdims=True)
    h = hc * jax.lax.rsqrt(var + 1e-5) * gn_ref[:] + btn_ref[:]
    h = jnp.maximum(h, 0.0)
    out = jnp.dot(h, wn3t_ref[:], preferred_element_type=f32) + bn3_ref[:]
    out_ref[:] = out.reshape(O, G, D)


@functools.partial(jax.jit, static_argnames=("G", "interpret"))
def _run(states, action, We1, be1, We2, be2, ge, bte, We3, be3,
         Wn1, bn1, Wn2, bn2, gn, btn, Wn3, bn3, *, G=512, interpret=False):
    Bv, O, D = states.shape
    H = We1.shape[0]
    A = Wn1.shape[1] - H - D
    assert Bv % G == 0
    grid = Bv // G

    xs = states.transpose(1, 0, 2)
    a2 = action.astype(jnp.int32).reshape(Bv, 1)
    w1uv = jnp.concatenate([We1[:, :D].T, We1[:, D:].T], axis=1)
    we2t = We2.T
    we3t = We3.T
    wn1at = Wn1[:, :D].T
    wn1bt = Wn1[:, D:D + A].T
    wn1ct = Wn1[:, D + A:].T
    wn2t = Wn2.T
    wn3t = Wn3.T
    row = lambda z: z.reshape(1, -1)

    full = lambda arr: pl.BlockSpec(arr.shape, lambda i: (0,) * arr.ndim)
    kern = functools.partial(_gnn_block_kernel, G=G, O=O, D=D, H=H, A=A)
    out = pl.pallas_call(
        kern,
        grid=(grid,),
        in_specs=[
            pl.BlockSpec((O, G, D), lambda i: (0, i, 0)),
            pl.BlockSpec((G, 1), lambda i: (i, 0)),
            full(w1uv), full(row(be1)), full(we2t), full(row(be2)),
            full(row(ge)), full(row(bte)), full(we3t), full(row(be3)),
            full(wn1at), full(wn1bt), full(wn1ct), full(row(bn1)),
            full(wn2t), full(row(bn2)), full(row(gn)), full(row(btn)),
            full(wn3t), full(row(bn3)),
        ],
        out_specs=pl.BlockSpec((O, G, D), lambda i: (0, i, 0)),
        out_shape=jax.ShapeDtypeStruct((O, Bv, D), jnp.float32),
        interpret=interpret,
    )(
        xs, a2, w1uv, row(be1), we2t, row(be2), row(ge), row(bte),
        we3t, row(be3), wn1at, wn1bt, wn1ct, row(bn1), wn2t, row(bn2),
        row(gn), row(btn), wn3t, row(bn3),
    )
    return out.transpose(1, 0, 2)


def kernel(states, action, We1, be1, We2, be2, ge, bte, We3, be3,
           Wn1, bn1, Wn2, bn2, gn, btn, Wn3, bn3):
    return _run(states, action, We1, be1, We2, be2, ge, bte, We3, be3,
                Wn1, bn1, Wn2, bn2, gn, btn, Wn3, bn3)


# speedup vs baseline: 14.4741x; 1.0054x over previous
import functools

import jax
import jax.numpy as jnp
from jax.experimental import pallas as pl


def _gnn_block_kernel(
    xs_ref, a_ref, w1uv_ref, be1_ref, we2t_ref, be2_ref, ge_ref, bte_ref,
    we3t_ref, be3_ref, wn1at_ref, wn1bt_ref, wn1ct_ref, bn1_ref, wn2t_ref,
    bn2_ref, gn_ref, btn_ref, wn3t_ref, bn3_ref, out_ref, *, G, O, D, H, A,
):
    f32 = jnp.float32
    x = xs_ref[:]
    x2 = x.reshape(O * G, D)

    uv = jnp.dot(x2, w1uv_ref[:], preferred_element_type=f32)
    u = uv[:, :H].reshape(O, 1, G, H)
    v = uv[:, H:].reshape(1, O, G, H)
    p = jnp.maximum(u + v + be1_ref[:].reshape(1, 1, 1, H), 0.0)
    p = p.reshape(O * O * G, H)

    h = jnp.dot(p, we2t_ref[:], preferred_element_type=f32) + be2_ref[:]
    m = jnp.mean(h, axis=-1, keepdims=True)
    hc = h - m
    var = jnp.mean(hc * hc, axis=-1, keepdims=True)
    h = hc * jax.lax.rsqrt(var + 1e-5) * ge_ref[:] + bte_ref[:]
    h = jnp.maximum(h, 0.0)
    e3 = jnp.dot(h, we3t_ref[:], preferred_element_type=f32) + be3_ref[:]

    e4 = e3.reshape(O, O, G, H)
    diag = jnp.stack([e4[i, i] for i in range(O)], axis=0)
    agg = (jnp.sum(e4, axis=1) - diag).reshape(O * G, H)

    a = a_ref[:]
    onehot = (a == jax.lax.broadcasted_iota(jnp.int32, (1, A), 1)).astype(f32)
    acth = jnp.dot(onehot, wn1bt_ref[:], preferred_element_type=f32)
    acth = jnp.broadcast_to(acth.reshape(1, G, H), (O, G, H)).reshape(O * G, H)
    t = (jnp.dot(x2, wn1at_ref[:], preferred_element_type=f32)
         + acth
         + jnp.dot(agg, wn1ct_ref[:], preferred_element_type=f32)
         + bn1_ref[:])
    t = jnp.maximum(t, 0.0)
    h = jnp.dot(t, wn2t_ref[:], preferred_element_type=f32) + bn2_ref[:]
    m = jnp.mean(h, axis=-1, keepdims=True)
    hc = h - m
    var = jnp.mean(hc * hc, axis=-1, keepdims=True)
    h = hc * jax.lax.rsqrt(var + 1e-5) * gn_ref[:] + btn_ref[:]
    h = jnp.maximum(h, 0.0)
    out = jnp.dot(h, wn3t_ref[:], preferred_element_type=f32) + bn3_ref[:]
    out_ref[:] = out.reshape(O, G, D)


@functools.partial(jax.jit, static_argnames=("G", "interpret"))
def _run(states, action, We1, be1, We2, be2, ge, bte, We3, be3,
         Wn1, bn1, Wn2, bn2, gn, btn, Wn3, bn3, *, G=1024, interpret=False):
    Bv, O, D = states.shape
    H = We1.shape[0]
    A = Wn1.shape[1] - H - D
    assert Bv % G == 0
    grid = Bv // G

    xs = states.transpose(1, 0, 2)
    a2 = action.astype(jnp.int32).reshape(Bv, 1)
    w1uv = jnp.concatenate([We1[:, :D].T, We1[:, D:].T], axis=1)
    we2t = We2.T
    we3t = We3.T
    wn1at = Wn1[:, :D].T
    wn1bt = Wn1[:, D:D + A].T
    wn1ct = Wn1[:, D + A:].T
    wn2t = Wn2.T
    wn3t = Wn3.T
    row = lambda z: z.reshape(1, -1)

    full = lambda arr: pl.BlockSpec(arr.shape, lambda i: (0,) * arr.ndim)
    kern = functools.partial(_gnn_block_kernel, G=G, O=O, D=D, H=H, A=A)
    out = pl.pallas_call(
        kern,
        grid=(grid,),
        in_specs=[
            pl.BlockSpec((O, G, D), lambda i: (0, i, 0)),
            pl.BlockSpec((G, 1), lambda i: (i, 0)),
            full(w1uv), full(row(be1)), full(we2t), full(row(be2)),
            full(row(ge)), full(row(bte)), full(we3t), full(row(be3)),
            full(wn1at), full(wn1bt), full(wn1ct), full(row(bn1)),
            full(wn2t), full(row(bn2)), full(row(gn)), full(row(btn)),
            full(wn3t), full(row(bn3)),
        ],
        out_specs=pl.BlockSpec((O, G, D), lambda i: (0, i, 0)),
        out_shape=jax.ShapeDtypeStruct((O, Bv, D), jnp.float32),
        interpret=interpret,
    )(
        xs, a2, w1uv, row(be1), we2t, row(be2), row(ge), row(bte),
        we3t, row(be3), wn1at, wn1bt, wn1ct, row(bn1), wn2t, row(bn2),
        row(gn), row(btn), wn3t, row(bn3),
    )
    return out.transpose(1, 0, 2)


def kernel(states, action, We1, be1, We2, be2, ge, bte, We3, be3,
           Wn1, bn1, Wn2, bn2, gn, btn, Wn3, bn3):
    return _run(states, action, We1, be1, We2, be2, ge, bte, We3, be3,
                Wn1, bn1, Wn2, bn2, gn, btn, Wn3, bn3)
